# baseline (speedup 1.0000x reference)
import functools

import jax
import jax.numpy as jnp
from jax import lax
from jax.experimental import pallas as pl
from jax.experimental.pallas import tpu as pltpu

N_NODES_C = 10000
N_EDGES_C = 8192
LOG2 = 0.6931471805599453


def _leaky(x):
    return jnp.where(x >= 0, x, 0.01 * x)


def _contrast_body(nm_ref, em_ref, out_ref, rs_ref, cs_ref, d_ref, *, bj, e):
    j = pl.program_id(0)
    nj = pl.num_programs(0)
    nm = nm_ref[...]
    em = em_ref[...]
    s = lax.dot_general(nm, em, (((1,), (1,)), ((), ())),
                        preferred_element_type=jnp.float32)
    z = jnp.exp(-jnp.abs(s))

    @pl.when(j == 0)
    def _init():
        rs_ref[...] = jnp.zeros_like(rs_ref)
        d_ref[...] = jnp.zeros_like(d_ref)

    rs_ref[0, :] += jnp.sum(z, axis=1)
    cs_ref[0, pl.ds(j * bj, bj)] = jnp.sum(z, axis=0)

    rows = lax.broadcasted_iota(jnp.int32, (e, bj), 0)
    cols = lax.broadcasted_iota(jnp.int32, (e, bj), 1) + j * bj
    d_ref[0, :] += jnp.sum(jnp.where(rows == cols, s, 0.0), axis=1)

    @pl.when(j == nj - 1)
    def _fin():
        out_ref[0, :] = (jnp.abs(d_ref[0, :]) - LOG2
                         + jnp.log(rs_ref[0, :] + cs_ref[0, :]))


def _contrast(nm, em, *, bj=512, interpret=False):
    e = nm.shape[0]
    nj = e // bj
    body = functools.partial(_contrast_body, bj=bj, e=e)
    out = pl.pallas_call(
        body,
        grid=(nj,),
        in_specs=[
            pl.BlockSpec((e, nm.shape[1]), lambda j: (0, 0)),
            pl.BlockSpec((bj, em.shape[1]), lambda j: (j, 0)),
        ],
        out_specs=pl.BlockSpec((1, e), lambda j: (0, 0)),
        out_shape=jax.ShapeDtypeStruct((1, e), jnp.float32),
        scratch_shapes=[
            pltpu.VMEM((1, e), jnp.float32),
            pltpu.VMEM((1, e), jnp.float32),
            pltpu.VMEM((1, e), jnp.float32),
        ],
        interpret=interpret,
    )(nm, em)
    return out[0]


def _gcn_conv(x, edge_index, w, b, num_nodes):
    x = x @ w
    loop = jnp.arange(num_nodes, dtype=edge_index.dtype)
    row = jnp.concatenate([edge_index[0], loop])
    col = jnp.concatenate([edge_index[1], loop])
    deg = jax.ops.segment_sum(jnp.ones_like(col, dtype=x.dtype), col,
                              num_segments=num_nodes)
    dinv = jnp.where(deg > 0, 1.0 / jnp.sqrt(deg), 0.0)
    norm = dinv[row] * dinv[col]
    out = jax.ops.segment_sum(norm[:, None] * x[row], col,
                              num_segments=num_nodes)
    return out + b


def _hyper_conv(x, hyperedge_index, w, b, num_nodes, num_hyperedges):
    x = x @ w
    node_idx = hyperedge_index[0]
    he_idx = hyperedge_index[1]
    d = jax.ops.segment_sum(jnp.ones_like(node_idx, dtype=x.dtype), node_idx,
                            num_segments=num_nodes)
    dinv = jnp.where(d > 0, 1.0 / d, 0.0)
    bb = jax.ops.segment_sum(jnp.ones_like(he_idx, dtype=x.dtype), he_idx,
                             num_segments=num_hyperedges)
    binv = jnp.where(bb > 0, 1.0 / bb, 0.0)
    he_feat = jax.ops.segment_sum(binv[he_idx][:, None] * x[node_idx], he_idx,
                                  num_segments=num_hyperedges)
    out = jax.ops.segment_sum(dinv[node_idx][:, None] * he_feat[he_idx],
                              node_idx, num_segments=num_nodes)
    return out + b


def kernel(nodes_feature, edges_feature, edge_index, hyperedge_index,
           gcn_w1, gcn_b1, gcn_w2, gcn_b2,
           hgc_w1, hgc_b1, hgc_w2, hgc_b2,
           node_w, node_b, edge_w, edge_b):
    n_nodes = nodes_feature.shape[0]
    n_edges = edges_feature.shape[0]
    e = edge_index.shape[1]

    h = _leaky(_gcn_conv(nodes_feature, edge_index, gcn_w1, gcn_b1, n_nodes))
    nodes_embedding = _leaky(_gcn_conv(h, edge_index, gcn_w2, gcn_b2, n_nodes))

    g = _leaky(_hyper_conv(edges_feature, hyperedge_index, hgc_w1, hgc_b1,
                           n_edges, 8192))
    edges_embedding = _leaky(_hyper_conv(g, hyperedge_index, hgc_w2, hgc_b2,
                                         n_edges, 8192))

    sel = nodes_embedding[edge_index.reshape(-1)]
    nodes_concat = jnp.concatenate([sel[:e], sel[e:]], axis=-1)
    nodes_map = nodes_concat @ node_w + node_b
    edges_map = edges_embedding @ edge_w + edge_b
    nm = nodes_map / jnp.linalg.norm(nodes_map, axis=-1, keepdims=True)
    em = edges_map / jnp.linalg.norm(edges_map, axis=-1, keepdims=True)
    return _contrast(nm, em)


# speedup vs baseline: 3.6668x; 3.6668x over previous
import functools

import jax
import jax.numpy as jnp
from jax import lax
from jax.experimental import pallas as pl
from jax.experimental.pallas import tpu as pltpu
from jax.experimental.pallas import tpu_sc as plsc

N_NODES = 10000
N_NODES_PAD = 10112
N_EDGES = 8192
HE_NNZ = 32768
FEAT = 128
MAP = 64
LOG2 = 0.6931471805599453

NC = 2
NS = 16
NW = NC * NS


def _leaky(x):
    return jnp.where(x >= 0, x, 0.01 * x)


def _sc_mesh():
    return plsc.VectorSubcoreMesh(core_axis_name="c", subcore_axis_name="s")


def _zero_table(zeros_hbm, table, sid, rows_per_tile):
    r0 = sid * rows_per_tile
    pltpu.sync_copy(zeros_hbm.at[pl.ds(r0, rows_per_tile)],
                    table.at[pl.ds(r0, rows_per_tile)])


def _writeback(table, out_hbm, cid, sid, rows_per_tile):
    r0 = sid * rows_per_tile
    pltpu.sync_copy(table.at[pl.ds(r0, rows_per_tile)],
                    out_hbm.at[cid, pl.ds(r0, rows_per_tile)])


def _scatter_pass(src_hbm, gidx_hbm, sidx_hbm, table, idx_v, rows_v, sem,
                  wid, n_items, chunk):
    per_tile = n_items // NW
    base = wid * per_tile
    for c in range(per_tile // chunk):
        off = base + c * chunk
        pltpu.sync_copy(gidx_hbm.at[pl.ds(off, chunk)], idx_v)
        pltpu.async_copy(src_hbm.at[idx_v], rows_v, sem).wait()
        pltpu.sync_copy(sidx_hbm.at[pl.ds(off, chunk)], idx_v)
        pltpu.sync_copy(rows_v, table.at[idx_v], add=True)


def _hist_body(col_hbm, node_hbm, he_hbm, ones_hbm, zeros_hbm,
               hc_out, hd_out, hb_out,
               table, idx_v, ones_v):
    cid = lax.axis_index("c")
    sid = lax.axis_index("s")
    wid = sid * NC + cid
    pltpu.sync_copy(ones_hbm, ones_v)

    _zero_table(zeros_hbm, table, sid, N_NODES_PAD // NS)
    plsc.subcore_barrier()
    pltpu.sync_copy(col_hbm.at[pl.ds(wid * 256, 256)], idx_v)
    pltpu.sync_copy(ones_v, table.at[idx_v], add=True)
    plsc.subcore_barrier()
    _writeback(table, hc_out, cid, sid, N_NODES_PAD // NS)
    plsc.subcore_barrier()

    _zero_table(zeros_hbm, table, sid, N_EDGES // NS)
    plsc.subcore_barrier()
    for c in range(4):
        off = wid * 1024 + c * 256
        pltpu.sync_copy(node_hbm.at[pl.ds(off, 256)], idx_v)
        pltpu.sync_copy(ones_v, table.at[idx_v], add=True)
    plsc.subcore_barrier()
    _writeback(table, hd_out, cid, sid, N_EDGES // NS)
    plsc.subcore_barrier()

    _zero_table(zeros_hbm, table, sid, N_EDGES // NS)
    plsc.subcore_barrier()
    for c in range(4):
        off = wid * 1024 + c * 256
        pltpu.sync_copy(he_hbm.at[pl.ds(off, 256)], idx_v)
        pltpu.sync_copy(ones_v, table.at[idx_v], add=True)
    plsc.subcore_barrier()
    _writeback(table, hb_out, cid, sid, N_EDGES // NS)


def _sc_hist(col_idx, node_idx, he_idx, ones128, zeros128):
    f32 = jnp.float32
    fn = pl.kernel(
        _hist_body,
        mesh=_sc_mesh(),
        out_type=[
            jax.ShapeDtypeStruct((NC, N_NODES_PAD, FEAT), f32),
            jax.ShapeDtypeStruct((NC, N_EDGES, FEAT), f32),
            jax.ShapeDtypeStruct((NC, N_EDGES, FEAT), f32),
        ],
        scratch_types=[
            pltpu.VMEM_SHARED((N_NODES_PAD, FEAT), f32),
            pltpu.VMEM((256,), jnp.int32),
            pltpu.VMEM((256, FEAT), f32),
        ],
    )
    return fn(col_idx, node_idx, he_idx, ones128, zeros128)


def _pair_body(xs_hbm, g_hbm, row_hbm, col_hbm, node_hbm, he_hbm, zeros_hbm,
               accg_out, acch_out,
               table, idx_v, rows_v, sem, *, n_g_rows):
    cid = lax.axis_index("c")
    sid = lax.axis_index("s")
    wid = sid * NC + cid

    _zero_table(zeros_hbm, table, sid, n_g_rows // NS)
    plsc.subcore_barrier()
    _scatter_pass(xs_hbm, row_hbm, col_hbm, table, idx_v, rows_v, sem,
                  wid, N_EDGES, 256)
    plsc.subcore_barrier()
    _writeback(table, accg_out, cid, sid, n_g_rows // NS)
    plsc.subcore_barrier()

    _zero_table(zeros_hbm, table, sid, N_EDGES // NS)
    plsc.subcore_barrier()
    _scatter_pass(g_hbm, node_hbm, he_hbm, table, idx_v, rows_v, sem,
                  wid, HE_NNZ, 256)
    plsc.subcore_barrier()
    _writeback(table, acch_out, cid, sid, N_EDGES // NS)


def _sc_pair(xs, g, gather_idx1, scatter_idx1, gather_idx2, scatter_idx2,
             zeros128, n_g_rows):
    f32 = jnp.float32
    body = functools.partial(_pair_body, n_g_rows=n_g_rows)
    fn = pl.kernel(
        body,
        mesh=_sc_mesh(),
        out_type=[
            jax.ShapeDtypeStruct((NC, n_g_rows, FEAT), f32),
            jax.ShapeDtypeStruct((NC, N_EDGES, FEAT), f32),
        ],
        scratch_types=[
            pltpu.VMEM_SHARED((n_g_rows, FEAT), f32),
            pltpu.VMEM((256,), jnp.int32),
            pltpu.VMEM((256, FEAT), f32),
            pltpu.SemaphoreType.DMA,
        ],
    )
    return fn(xs, g, gather_idx1, scatter_idx1, gather_idx2, scatter_idx2,
              zeros128)


def _single_body(src_hbm, gidx_hbm, sidx_hbm, p_hbm, q_hbm, row_hbm, col_hbm,
                 zeros_hbm,
                 acc_out, pg_out, qg_out,
                 table, idx_v, rows_v, sem):
    cid = lax.axis_index("c")
    sid = lax.axis_index("s")
    wid = sid * NC + cid

    _zero_table(zeros_hbm, table, sid, N_EDGES // NS)
    plsc.subcore_barrier()
    _scatter_pass(src_hbm, gidx_hbm, sidx_hbm, table, idx_v, rows_v, sem,
                  wid, HE_NNZ, 256)

    base = wid * 256
    pltpu.sync_copy(row_hbm.at[pl.ds(base, 256)], idx_v)
    pltpu.async_copy(p_hbm.at[idx_v], rows_v, sem).wait()
    pltpu.sync_copy(rows_v, pg_out.at[pl.ds(base, 256)])
    pltpu.sync_copy(col_hbm.at[pl.ds(base, 256)], idx_v)
    pltpu.async_copy(q_hbm.at[idx_v], rows_v, sem).wait()
    pltpu.sync_copy(rows_v, qg_out.at[pl.ds(base, 256)])

    plsc.subcore_barrier()
    _writeback(table, acc_out, cid, sid, N_EDGES // NS)


def _sc_hyp_and_gather(src, gidx, sidx, p, q, row_idx, col_idx, zeros128):
    f32 = jnp.float32
    fn = pl.kernel(
        _single_body,
        mesh=_sc_mesh(),
        out_type=[
            jax.ShapeDtypeStruct((NC, N_EDGES, FEAT), f32),
            jax.ShapeDtypeStruct((N_EDGES, FEAT), f32),
            jax.ShapeDtypeStruct((N_EDGES, FEAT), f32),
        ],
        scratch_types=[
            pltpu.VMEM_SHARED((N_EDGES, FEAT), f32),
            pltpu.VMEM((256,), jnp.int32),
            pltpu.VMEM((256, FEAT), f32),
            pltpu.SemaphoreType.DMA,
        ],
    )
    return fn(src, gidx, sidx, p, q, row_idx, col_idx, zeros128)


def _last_body(src_hbm, gidx_hbm, sidx_hbm, zeros_hbm, acc_out,
               table, idx_v, rows_v, sem):
    cid = lax.axis_index("c")
    sid = lax.axis_index("s")
    wid = sid * NC + cid
    _zero_table(zeros_hbm, table, sid, N_EDGES // NS)
    plsc.subcore_barrier()
    _scatter_pass(src_hbm, gidx_hbm, sidx_hbm, table, idx_v, rows_v, sem,
                  wid, HE_NNZ, 256)
    plsc.subcore_barrier()
    _writeback(table, acc_out, cid, sid, N_EDGES // NS)


def _sc_hyp(src, gidx, sidx, zeros128):
    f32 = jnp.float32
    fn = pl.kernel(
        _last_body,
        mesh=_sc_mesh(),
        out_type=jax.ShapeDtypeStruct((NC, N_EDGES, FEAT), f32),
        scratch_types=[
            pltpu.VMEM_SHARED((N_EDGES, FEAT), f32),
            pltpu.VMEM((256,), jnp.int32),
            pltpu.VMEM((256, FEAT), f32),
            pltpu.SemaphoreType.DMA,
        ],
    )
    return fn(src, gidx, sidx, zeros128)


def _mm1_body(nodes_ref, w1_ref, edges_ref, wh_ref, xw_ref, g_ref):
    xw_ref[...] = jnp.dot(nodes_ref[...], w1_ref[...],
                          preferred_element_type=jnp.float32)
    g_ref[...] = jnp.dot(edges_ref[...], wh_ref[...],
                         preferred_element_type=jnp.float32)


def _tc_mm1(nodes, w1, edges, wh):
    return pl.pallas_call(
        _mm1_body,
        out_shape=[jax.ShapeDtypeStruct((N_NODES, FEAT), jnp.float32),
                   jax.ShapeDtypeStruct((N_EDGES, FEAT), jnp.float32)],
    )(nodes, w1, edges, wh)


def _dinv_from_hist(hc_ref):
    h = hc_ref[0, :, 0] + hc_ref[1, :, 0] + 1.0
    return (1.0 / jnp.sqrt(h))[:N_NODES, None]


def _recip_from_hist(hr_ref):
    h = hr_ref[0, :, 0] + hr_ref[1, :, 0]
    return jnp.where(h > 0, 1.0 / h, 0.0)[:, None]


def _scale_body(hc_ref, xw_ref, xs_ref):
    xs_ref[...] = _dinv_from_hist(hc_ref) * xw_ref[...]


def _tc_scale(hc, xw):
    return pl.pallas_call(
        _scale_body,
        out_shape=jax.ShapeDtypeStruct((N_NODES, FEAT), jnp.float32),
    )(hc, xw)


def _l1fin_body(hc_ref, accg_ref, xs1_ref, b1_ref, w2_ref,
                hb_ref, acch_ref, xs2_ref, he1_ref):
    dinv = _dinv_from_hist(hc_ref)
    acc = accg_ref[0, :N_NODES, :] + accg_ref[1, :N_NODES, :]
    h = _leaky(dinv * (acc + xs1_ref[...]) + b1_ref[...])
    xw2 = jnp.dot(h, w2_ref[...], preferred_element_type=jnp.float32)
    xs2_ref[...] = dinv * xw2
    binv = _recip_from_hist(hb_ref)
    he1_ref[...] = binv * (acch_ref[0] + acch_ref[1])


def _tc_l1fin(hc, accg, xs1, b1, w2, hb, acch):
    return pl.pallas_call(
        _l1fin_body,
        out_shape=[jax.ShapeDtypeStruct((N_NODES, FEAT), jnp.float32),
                   jax.ShapeDtypeStruct((N_EDGES, FEAT), jnp.float32)],
    )(hc, accg, xs1, b1[None, :], w2, hb, acch)


def _l2fin_a_body(hc_ref, accg_ref, xs2_ref, b2_ref, nwa_ref, nwb_ref,
                  p_ref, q_ref):
    dinv = _dinv_from_hist(hc_ref)
    acc = accg_ref[0, :N_NODES, :] + accg_ref[1, :N_NODES, :]
    ne = _leaky(dinv * (acc + xs2_ref[...]) + b2_ref[...])
    p_ref[...] = jnp.dot(ne, nwa_ref[...], preferred_element_type=jnp.float32)
    q_ref[...] = jnp.dot(ne, nwb_ref[...], preferred_element_type=jnp.float32)


def _l2fin_b_body(hd_ref, acch_ref, hb1_ref, hw2_ref, gw2_ref):
    dinv_h = _recip_from_hist(hd_ref)
    g2 = _leaky(dinv_h * (acch_ref[0] + acch_ref[1]) + hb1_ref[...])
    gw2_ref[...] = jnp.dot(g2, hw2_ref[...], preferred_element_type=jnp.float32)


def _tc_l2fin(hc, accg, xs2, b2, nwa, nwb, hd, acch, hb1, hw2):
    p, q = pl.pallas_call(
        _l2fin_a_body,
        out_shape=[jax.ShapeDtypeStruct((N_NODES, FEAT), jnp.float32),
                   jax.ShapeDtypeStruct((N_NODES, FEAT), jnp.float32)],
    )(hc, accg, xs2, b2[None, :], nwa, nwb)
    gw2 = pl.pallas_call(
        _l2fin_b_body,
        out_shape=jax.ShapeDtypeStruct((N_EDGES, FEAT), jnp.float32),
    )(hd, acch, hb1[None, :], hw2)
    return p, q, gw2


def _he2_body(hb_ref, acch_ref, he2_ref):
    he2_ref[...] = _recip_from_hist(hb_ref) * (acch_ref[0] + acch_ref[1])


def _tc_he2(hb, acch):
    return pl.pallas_call(
        _he2_body,
        out_shape=jax.ShapeDtypeStruct((N_EDGES, FEAT), jnp.float32),
    )(hb, acch)


def _maps_body(hd_ref, acch_ref, hb2_ref, ew_ref, eb_ref,
               pg_ref, qg_ref, nb_ref, nm_ref, em_ref):
    dinv_h = _recip_from_hist(hd_ref)
    ee = _leaky(dinv_h * (acch_ref[0] + acch_ref[1]) + hb2_ref[...])
    emap = jnp.dot(ee, ew_ref[...], preferred_element_type=jnp.float32)
    emap = emap + eb_ref[...]
    nmap = (pg_ref[...] + qg_ref[...])[:, :MAP] + nb_ref[...]
    nm_ref[...] = nmap * lax.rsqrt(jnp.sum(nmap * nmap, axis=1,
                                           keepdims=True))
    em_ref[...] = emap * lax.rsqrt(jnp.sum(emap * emap, axis=1,
                                           keepdims=True))


def _tc_maps(hd, acch, hb2, ew, eb, pg, qg, nb):
    return pl.pallas_call(
        _maps_body,
        out_shape=[jax.ShapeDtypeStruct((N_EDGES, MAP), jnp.float32),
                   jax.ShapeDtypeStruct((N_EDGES, MAP), jnp.float32)],
    )(hd, acch, hb2[None, :], ew, eb[None, :], pg, qg, nb[None, :])


def _contrast_body(nm_ref, em_ref, out_ref, rs_ref, cs_ref, d_ref, *, bj, e):
    j = pl.program_id(0)
    nj = pl.num_programs(0)
    nm = nm_ref[...]
    em = em_ref[...]
    s = lax.dot_general(nm, em, (((1,), (1,)), ((), ())),
                        preferred_element_type=jnp.float32)
    z = jnp.exp(-jnp.abs(s))

    @pl.when(j == 0)
    def _init():
        rs_ref[...] = jnp.zeros_like(rs_ref)
        d_ref[...] = jnp.zeros_like(d_ref)

    rs_ref[0, :] += jnp.sum(z, axis=1)
    cs_ref[0, pl.ds(j * bj, bj)] = jnp.sum(z, axis=0)

    rows = lax.broadcasted_iota(jnp.int32, (e, bj), 0)
    cols = lax.broadcasted_iota(jnp.int32, (e, bj), 1) + j * bj
    d_ref[0, :] += jnp.sum(jnp.where(rows == cols, s, 0.0), axis=1)

    @pl.when(j == nj - 1)
    def _fin():
        out_ref[0, :] = (jnp.abs(d_ref[0, :]) - LOG2
                         + jnp.log(rs_ref[0, :] + cs_ref[0, :]))


def _contrast(nm, em, *, bj=512, interpret=False):
    e = nm.shape[0]
    nj = e // bj
    body = functools.partial(_contrast_body, bj=bj, e=e)
    out = pl.pallas_call(
        body,
        grid=(nj,),
        in_specs=[
            pl.BlockSpec((e, nm.shape[1]), lambda j: (0, 0)),
            pl.BlockSpec((bj, em.shape[1]), lambda j: (j, 0)),
        ],
        out_specs=pl.BlockSpec((1, e), lambda j: (0, 0)),
        out_shape=jax.ShapeDtypeStruct((1, e), jnp.float32),
        scratch_shapes=[
            pltpu.VMEM((1, e), jnp.float32),
            pltpu.VMEM((1, e), jnp.float32),
            pltpu.VMEM((1, e), jnp.float32),
        ],
        interpret=interpret,
    )(nm, em)
    return out[0]


def kernel(nodes_feature, edges_feature, edge_index, hyperedge_index,
           gcn_w1, gcn_b1, gcn_w2, gcn_b2,
           hgc_w1, hgc_b1, hgc_w2, hgc_b2,
           node_w, node_b, edge_w, edge_b):
    f32 = jnp.float32
    row_idx = edge_index[0]
    col_idx = edge_index[1]
    node_idx = hyperedge_index[0]
    he_idx = hyperedge_index[1]

    ones128 = jnp.ones((256, FEAT), f32)
    zeros128 = jnp.zeros((N_NODES_PAD, FEAT), f32)

    hc, hd, hb = _sc_hist(col_idx, node_idx, he_idx, ones128, zeros128)

    xw1, g1 = _tc_mm1(nodes_feature, gcn_w1, edges_feature, hgc_w1)
    xs1 = _tc_scale(hc, xw1)

    accg1, acch1a = _sc_pair(xs1, g1, row_idx, col_idx, node_idx, he_idx,
                             zeros128, N_NODES_PAD)

    xs2, he1 = _tc_l1fin(hc, accg1, xs1, gcn_b1, gcn_w2, hb, acch1a)

    accg2, acch1b = _sc_pair(xs2, he1, row_idx, col_idx, he_idx, node_idx,
                             zeros128, N_NODES_PAD)

    wpad = jnp.zeros((FEAT, FEAT - MAP), f32)
    nwa = jnp.concatenate([node_w[:FEAT], wpad], axis=1)
    nwb = jnp.concatenate([node_w[FEAT:], wpad], axis=1)
    p, q, gw2 = _tc_l2fin(hc, accg2, xs2, gcn_b2, nwa, nwb, hd, acch1b,
                          hgc_b1, hgc_w2)

    acch2a, pg, qg = _sc_hyp_and_gather(gw2, node_idx, he_idx, p, q,
                                        row_idx, col_idx, zeros128)

    he2 = _tc_he2(hb, acch2a)

    acch2b = _sc_hyp(he2, he_idx, node_idx, zeros128)

    nm, em = _tc_maps(hd, acch2b, hgc_b2, edge_w, edge_b, pg, qg, node_b)
    return _contrast(nm, em)


# speedup vs baseline: 3.8590x; 1.0524x over previous
import functools

import jax
import jax.numpy as jnp
from jax import lax
from jax.experimental import pallas as pl
from jax.experimental.pallas import tpu as pltpu
from jax.experimental.pallas import tpu_sc as plsc

N_NODES = 10000
N_NODES_PAD = 10112
N_EDGES = 8192
HE_NNZ = 32768
FEAT = 128
MAP = 64
LOG2 = 0.6931471805599453

NC = 2
NS = 16


def _leaky(x):
    return jnp.where(x >= 0, x, 0.01 * x)


def _sc_mesh():
    return plsc.VectorSubcoreMesh(core_axis_name="c", subcore_axis_name="s")


def _zero_table(zeros_hbm, table, sid, rows_per_tile):
    r0 = sid * rows_per_tile
    pltpu.sync_copy(zeros_hbm.at[pl.ds(r0, rows_per_tile)],
                    table.at[pl.ds(r0, rows_per_tile)])


def _writeback(table, out_hbm, sid, rows_per_tile):
    r0 = sid * rows_per_tile
    pltpu.sync_copy(table.at[pl.ds(r0, rows_per_tile)],
                    out_hbm.at[pl.ds(r0, rows_per_tile)])


def _scatter_tile(src_hbm, gidx_hbm, sidx_hbm, table, idx_v, rows_v, sem,
                  sid, n_items):
    per_tile = n_items // NS
    for c in range(per_tile // 256):
        off = sid * per_tile + c * 256
        pltpu.sync_copy(gidx_hbm.at[pl.ds(off, 256)], idx_v)
        pltpu.async_copy(src_hbm.at[idx_v], rows_v, sem).wait()
        pltpu.sync_copy(sidx_hbm.at[pl.ds(off, 256)], idx_v)
        pltpu.sync_copy(rows_v, table.at[idx_v], add=True)


def _ones_scatter_tile(idx_hbm, table, idx_v, ones_v, sid, n_items):
    per_tile = n_items // NS
    for c in range(per_tile // 256):
        off = sid * per_tile + c * 256
        pltpu.sync_copy(idx_hbm.at[pl.ds(off, 256)], idx_v)
        pltpu.sync_copy(ones_v, table.at[idx_v], add=True)


def _hist_body(col_hbm, node_hbm, he_hbm, ones_hbm, zeros_hbm,
               hc_out, hd_out, hb_out,
               table, idx_v, ones_v):
    cid = lax.axis_index("c")
    sid = lax.axis_index("s")
    pltpu.sync_copy(ones_hbm, ones_v)

    @pl.when(cid == 0)
    def _core0():
        _zero_table(zeros_hbm, table, sid, N_NODES_PAD // NS)
        plsc.subcore_barrier()
        _ones_scatter_tile(col_hbm, table, idx_v, ones_v, sid, N_EDGES)
        plsc.subcore_barrier()
        _writeback(table, hc_out, sid, N_NODES_PAD // NS)
        plsc.subcore_barrier()
        _zero_table(zeros_hbm, table, sid, N_EDGES // NS)
        plsc.subcore_barrier()
        _ones_scatter_tile(node_hbm, table, idx_v, ones_v, sid, HE_NNZ)
        plsc.subcore_barrier()
        _writeback(table, hd_out, sid, N_EDGES // NS)

    @pl.when(cid == 1)
    def _core1():
        _zero_table(zeros_hbm, table, sid, N_EDGES // NS)
        plsc.subcore_barrier()
        _ones_scatter_tile(he_hbm, table, idx_v, ones_v, sid, HE_NNZ)
        plsc.subcore_barrier()
        _writeback(table, hb_out, sid, N_EDGES // NS)


def _sc_hist(col_idx, node_idx, he_idx, ones128, zeros128):
    f32 = jnp.float32
    fn = pl.kernel(
        _hist_body,
        mesh=_sc_mesh(),
        out_type=[
            jax.ShapeDtypeStruct((N_NODES_PAD, FEAT), f32),
            jax.ShapeDtypeStruct((N_EDGES, FEAT), f32),
            jax.ShapeDtypeStruct((N_EDGES, FEAT), f32),
        ],
        scratch_types=[
            pltpu.VMEM_SHARED((N_NODES_PAD, FEAT), f32),
            pltpu.VMEM((256,), jnp.int32),
            pltpu.VMEM((256, FEAT), f32),
        ],
    )
    return fn(col_idx, node_idx, he_idx, ones128, zeros128)


def _pair_body(xs_hbm, g_hbm, row_hbm, col_hbm, gi2_hbm, si2_hbm, zeros_hbm,
               accg_out, acch_out,
               table, idx_v, rows_v, sem):
    cid = lax.axis_index("c")
    sid = lax.axis_index("s")

    @pl.when(cid == 0)
    def _core0():
        _zero_table(zeros_hbm, table, sid, N_NODES_PAD // NS)
        plsc.subcore_barrier()
        _scatter_tile(xs_hbm, row_hbm, col_hbm, table, idx_v, rows_v, sem,
                      sid, N_EDGES)
        plsc.subcore_barrier()
        _writeback(table, accg_out, sid, N_NODES_PAD // NS)

    @pl.when(cid == 1)
    def _core1():
        _zero_table(zeros_hbm, table, sid, N_EDGES // NS)
        plsc.subcore_barrier()
        _scatter_tile(g_hbm, gi2_hbm, si2_hbm, table, idx_v, rows_v, sem,
                      sid, HE_NNZ)
        plsc.subcore_barrier()
        _writeback(table, acch_out, sid, N_EDGES // NS)


def _sc_pair(xs, g, gather_idx1, scatter_idx1, gather_idx2, scatter_idx2,
             zeros128):
    f32 = jnp.float32
    fn = pl.kernel(
        _pair_body,
        mesh=_sc_mesh(),
        out_type=[
            jax.ShapeDtypeStruct((N_NODES_PAD, FEAT), f32),
            jax.ShapeDtypeStruct((N_EDGES, FEAT), f32),
        ],
        scratch_types=[
            pltpu.VMEM_SHARED((N_NODES_PAD, FEAT), f32),
            pltpu.VMEM((256,), jnp.int32),
            pltpu.VMEM((256, FEAT), f32),
            pltpu.SemaphoreType.DMA,
        ],
    )
    return fn(xs, g, gather_idx1, scatter_idx1, gather_idx2, scatter_idx2,
              zeros128)


def _single_body(src_hbm, gidx_hbm, sidx_hbm, p_hbm, q_hbm, row_hbm, col_hbm,
                 zeros_hbm,
                 acc_out, pg_out, qg_out,
                 table, idx_v, rows_v, sem):
    cid = lax.axis_index("c")
    sid = lax.axis_index("s")

    @pl.when(cid == 0)
    def _core0():
        _zero_table(zeros_hbm, table, sid, N_EDGES // NS)
        plsc.subcore_barrier()
        _scatter_tile(src_hbm, gidx_hbm, sidx_hbm, table, idx_v, rows_v, sem,
                      sid, HE_NNZ)
        plsc.subcore_barrier()
        _writeback(table, acc_out, sid, N_EDGES // NS)

    @pl.when(cid == 1)
    def _core1():
        for c in range(2):
            base = sid * 512 + c * 256
            pltpu.sync_copy(row_hbm.at[pl.ds(base, 256)], idx_v)
            pltpu.async_copy(p_hbm.at[idx_v], rows_v, sem).wait()
            pltpu.sync_copy(rows_v, pg_out.at[pl.ds(base, 256)])
            pltpu.sync_copy(col_hbm.at[pl.ds(base, 256)], idx_v)
            pltpu.async_copy(q_hbm.at[idx_v], rows_v, sem).wait()
            pltpu.sync_copy(rows_v, qg_out.at[pl.ds(base, 256)])


def _sc_hyp_and_gather(src, gidx, sidx, p, q, row_idx, col_idx, zeros128):
    f32 = jnp.float32
    fn = pl.kernel(
        _single_body,
        mesh=_sc_mesh(),
        out_type=[
            jax.ShapeDtypeStruct((N_EDGES, FEAT), f32),
            jax.ShapeDtypeStruct((N_EDGES, FEAT), f32),
            jax.ShapeDtypeStruct((N_EDGES, FEAT), f32),
        ],
        scratch_types=[
            pltpu.VMEM_SHARED((N_EDGES, FEAT), f32),
            pltpu.VMEM((256,), jnp.int32),
            pltpu.VMEM((256, FEAT), f32),
            pltpu.SemaphoreType.DMA,
        ],
    )
    return fn(src, gidx, sidx, p, q, row_idx, col_idx, zeros128)


def _last_body(src_hbm, gidx_hbm, sidx_hbm, zeros_hbm, acc_out,
               table, idx_v, rows_v, sem):
    cid = lax.axis_index("c")
    sid = lax.axis_index("s")
    wid = sid * NC + cid
    _zero_table(zeros_hbm, table, sid, N_EDGES // NS)
    plsc.subcore_barrier()
    for c in range(4):
        off = wid * 1024 + c * 256
        pltpu.sync_copy(gidx_hbm.at[pl.ds(off, 256)], idx_v)
        pltpu.async_copy(src_hbm.at[idx_v], rows_v, sem).wait()
        pltpu.sync_copy(sidx_hbm.at[pl.ds(off, 256)], idx_v)
        pltpu.sync_copy(rows_v, table.at[idx_v], add=True)
    plsc.subcore_barrier()
    r0 = sid * (N_EDGES // NS)
    pltpu.sync_copy(table.at[pl.ds(r0, N_EDGES // NS)],
                    acc_out.at[cid, pl.ds(r0, N_EDGES // NS)])


def _sc_hyp(src, gidx, sidx, zeros128):
    f32 = jnp.float32
    fn = pl.kernel(
        _last_body,
        mesh=_sc_mesh(),
        out_type=jax.ShapeDtypeStruct((NC, N_EDGES, FEAT), f32),
        scratch_types=[
            pltpu.VMEM_SHARED((N_EDGES, FEAT), f32),
            pltpu.VMEM((256,), jnp.int32),
            pltpu.VMEM((256, FEAT), f32),
            pltpu.SemaphoreType.DMA,
        ],
    )
    return fn(src, gidx, sidx, zeros128)


def _dinv_from_hist(hc_ref):
    h = hc_ref[:, 0] + 1.0
    return (1.0 / jnp.sqrt(h))[:N_NODES, None]


def _recip_from_hist(hr_ref):
    h = hr_ref[:, 0]
    return jnp.where(h > 0, 1.0 / h, 0.0)[:, None]


def _mm1_body(hc_ref, nodes_ref, w1_ref, edges_ref, wh_ref, xs_ref, g_ref):
    xw = jnp.dot(nodes_ref[...], w1_ref[...],
                 preferred_element_type=jnp.float32)
    xs_ref[...] = _dinv_from_hist(hc_ref) * xw
    g_ref[...] = jnp.dot(edges_ref[...], wh_ref[...],
                         preferred_element_type=jnp.float32)


def _tc_mm1(hc, nodes, w1, edges, wh):
    return pl.pallas_call(
        _mm1_body,
        out_shape=[jax.ShapeDtypeStruct((N_NODES, FEAT), jnp.float32),
                   jax.ShapeDtypeStruct((N_EDGES, FEAT), jnp.float32)],
    )(hc, nodes, w1, edges, wh)


def _l1fin_body(hc_ref, accg_ref, xs1_ref, b1_ref, w2_ref,
                hb_ref, acch_ref, xs2_ref, he1_ref):
    dinv = _dinv_from_hist(hc_ref)
    h = _leaky(dinv * (accg_ref[:N_NODES, :] + xs1_ref[...]) + b1_ref[...])
    xw2 = jnp.dot(h, w2_ref[...], preferred_element_type=jnp.float32)
    xs2_ref[...] = dinv * xw2
    he1_ref[...] = _recip_from_hist(hb_ref) * acch_ref[...]


def _tc_l1fin(hc, accg, xs1, b1, w2, hb, acch):
    return pl.pallas_call(
        _l1fin_body,
        out_shape=[jax.ShapeDtypeStruct((N_NODES, FEAT), jnp.float32),
                   jax.ShapeDtypeStruct((N_EDGES, FEAT), jnp.float32)],
    )(hc, accg, xs1, b1[None, :], w2, hb, acch)


def _l2fin_body(hc_ref, accg_ref, xs2_ref, b2_ref, nwa_ref, nwb_ref,
                hd_ref, acch_ref, hb1_ref, hw2_ref,
                p_ref, q_ref, gw2_ref):
    dinv = _dinv_from_hist(hc_ref)
    ne = _leaky(dinv * (accg_ref[:N_NODES, :] + xs2_ref[...]) + b2_ref[...])
    p_ref[...] = jnp.dot(ne, nwa_ref[...], preferred_element_type=jnp.float32)
    q_ref[...] = jnp.dot(ne, nwb_ref[...], preferred_element_type=jnp.float32)
    dinv_h = _recip_from_hist(hd_ref)
    g2 = _leaky(dinv_h * acch_ref[...] + hb1_ref[...])
    gw2_ref[...] = jnp.dot(g2, hw2_ref[...], preferred_element_type=jnp.float32)


def _tc_l2fin(hc, accg, xs2, b2, nwa, nwb, hd, acch, hb1, hw2):
    return pl.pallas_call(
        _l2fin_body,
        out_shape=[jax.ShapeDtypeStruct((N_NODES, FEAT), jnp.float32),
                   jax.ShapeDtypeStruct((N_NODES, FEAT), jnp.float32),
                   jax.ShapeDtypeStruct((N_EDGES, FEAT), jnp.float32)],
    )(hc, accg, xs2, b2[None, :], nwa, nwb, hd, acch, hb1[None, :], hw2)


def _he2_body(hb_ref, acch_ref, he2_ref):
    he2_ref[...] = _recip_from_hist(hb_ref) * acch_ref[...]


def _tc_he2(hb, acch):
    return pl.pallas_call(
        _he2_body,
        out_shape=jax.ShapeDtypeStruct((N_EDGES, FEAT), jnp.float32),
    )(hb, acch)


def _maps_body(hd_ref, acch_ref, hb2_ref, ew_ref, eb_ref,
               pg_ref, qg_ref, nb_ref, nm_ref, em_ref):
    dinv_h = _recip_from_hist(hd_ref)
    ee = _leaky(dinv_h * (acch_ref[0] + acch_ref[1]) + hb2_ref[...])
    emap = jnp.dot(ee, ew_ref[...], preferred_element_type=jnp.float32)
    emap = emap + eb_ref[...]
    nmap = (pg_ref[...] + qg_ref[...])[:, :MAP] + nb_ref[...]
    nm_ref[...] = nmap * lax.rsqrt(jnp.sum(nmap * nmap, axis=1,
                                           keepdims=True))
    em_ref[...] = emap * lax.rsqrt(jnp.sum(emap * emap, axis=1,
                                           keepdims=True))


def _tc_maps(hd, acch, hb2, ew, eb, pg, qg, nb):
    return pl.pallas_call(
        _maps_body,
        out_shape=[jax.ShapeDtypeStruct((N_EDGES, MAP), jnp.float32),
                   jax.ShapeDtypeStruct((N_EDGES, MAP), jnp.float32)],
    )(hd, acch, hb2[None, :], ew, eb[None, :], pg, qg, nb[None, :])


def _contrast_body(nm_ref, em_ref, out_ref, rs_ref, cs_ref, d_ref, *, bj, e):
    j = pl.program_id(0)
    nj = pl.num_programs(0)
    nm = nm_ref[...]
    em = em_ref[...]
    s = lax.dot_general(nm, em, (((1,), (1,)), ((), ())),
                        preferred_element_type=jnp.float32)
    z = jnp.exp(-jnp.abs(s))

    @pl.when(j == 0)
    def _init():
        rs_ref[...] = jnp.zeros_like(rs_ref)
        d_ref[...] = jnp.zeros_like(d_ref)

    rs_ref[0, :] += jnp.sum(z, axis=1)
    cs_ref[0, pl.ds(j * bj, bj)] = jnp.sum(z, axis=0)

    rows = lax.broadcasted_iota(jnp.int32, (e, bj), 0)
    cols = lax.broadcasted_iota(jnp.int32, (e, bj), 1) + j * bj
    d_ref[0, :] += jnp.sum(jnp.where(rows == cols, s, 0.0), axis=1)

    @pl.when(j == nj - 1)
    def _fin():
        out_ref[0, :] = (jnp.abs(d_ref[0, :]) - LOG2
                         + jnp.log(rs_ref[0, :] + cs_ref[0, :]))


def _contrast(nm, em, *, bj=512, interpret=False):
    e = nm.shape[0]
    nj = e // bj
    body = functools.partial(_contrast_body, bj=bj, e=e)
    out = pl.pallas_call(
        body,
        grid=(nj,),
        in_specs=[
            pl.BlockSpec((e, nm.shape[1]), lambda j: (0, 0)),
            pl.BlockSpec((bj, em.shape[1]), lambda j: (j, 0)),
        ],
        out_specs=pl.BlockSpec((1, e), lambda j: (0, 0)),
        out_shape=jax.ShapeDtypeStruct((1, e), jnp.float32),
        scratch_shapes=[
            pltpu.VMEM((1, e), jnp.float32),
            pltpu.VMEM((1, e), jnp.float32),
            pltpu.VMEM((1, e), jnp.float32),
        ],
        interpret=interpret,
    )(nm, em)
    return out[0]


def kernel(nodes_feature, edges_feature, edge_index, hyperedge_index,
           gcn_w1, gcn_b1, gcn_w2, gcn_b2,
           hgc_w1, hgc_b1, hgc_w2, hgc_b2,
           node_w, node_b, edge_w, edge_b):
    f32 = jnp.float32
    row_idx = edge_index[0]
    col_idx = edge_index[1]
    node_idx = hyperedge_index[0]
    he_idx = hyperedge_index[1]

    ones128 = jnp.ones((256, FEAT), f32)
    zeros128 = jnp.zeros((N_NODES_PAD, FEAT), f32)

    hc, hd, hb = _sc_hist(col_idx, node_idx, he_idx, ones128, zeros128)

    xs1, g1 = _tc_mm1(hc, nodes_feature, gcn_w1, edges_feature, hgc_w1)

    accg1, acch1a = _sc_pair(xs1, g1, row_idx, col_idx, node_idx, he_idx,
                             zeros128)

    xs2, he1 = _tc_l1fin(hc, accg1, xs1, gcn_b1, gcn_w2, hb, acch1a)

    accg2, acch1b = _sc_pair(xs2, he1, row_idx, col_idx, he_idx, node_idx,
                             zeros128)

    wpad = jnp.zeros((FEAT, FEAT - MAP), f32)
    nwa = jnp.concatenate([node_w[:FEAT], wpad], axis=1)
    nwb = jnp.concatenate([node_w[FEAT:], wpad], axis=1)
    p, q, gw2 = _tc_l2fin(hc, accg2, xs2, gcn_b2, nwa, nwb, hd, acch1b,
                          hgc_b1, hgc_w2)

    acch2a, pg, qg = _sc_hyp_and_gather(gw2, node_idx, he_idx, p, q,
                                        row_idx, col_idx, zeros128)

    he2 = _tc_he2(hb, acch2a)

    acch2b = _sc_hyp(he2, he_idx, node_idx, zeros128)

    nm, em = _tc_maps(hd, acch2b, hgc_b2, edge_w, edge_b, pg, qg, node_b)
    return _contrast(nm, em)


# speedup vs baseline: 4.5338x; 1.1749x over previous
import functools

import jax
import jax.numpy as jnp
from jax import lax
from jax.experimental import pallas as pl
from jax.experimental.pallas import tpu as pltpu
from jax.experimental.pallas import tpu_sc as plsc

N_NODES = 10000
N_NODES_PAD = 10112
N_EDGES = 8192
HE_NNZ = 32768
FEAT = 128
MAP = 64
LOG2 = 0.6931471805599453

NC = 2
NS = 16


def _leaky(x):
    return jnp.where(x >= 0, x, 0.01 * x)


def _sc_mesh():
    return plsc.VectorSubcoreMesh(core_axis_name="c", subcore_axis_name="s")


def _zero_table(zeros_hbm, table, sid, rows_per_tile):
    r0 = sid * rows_per_tile
    pltpu.sync_copy(zeros_hbm.at[pl.ds(r0, rows_per_tile)],
                    table.at[pl.ds(r0, rows_per_tile)])


def _writeback(table, out_hbm, sid, rows_per_tile):
    r0 = sid * rows_per_tile
    pltpu.sync_copy(table.at[pl.ds(r0, rows_per_tile)],
                    out_hbm.at[pl.ds(r0, rows_per_tile)])


def _scatter_tile(src_hbm, gidx_hbm, sidx_hbm, table, idx_v, rows_v, sem,
                  sid, n_items):
    per_tile = n_items // NS
    for c in range(per_tile // 256):
        off = sid * per_tile + c * 256
        pltpu.sync_copy(gidx_hbm.at[pl.ds(off, 256)], idx_v)
        pltpu.async_copy(src_hbm.at[idx_v], rows_v, sem).wait()
        pltpu.sync_copy(sidx_hbm.at[pl.ds(off, 256)], idx_v)
        pltpu.sync_copy(rows_v, table.at[idx_v], add=True)


def _ones_scatter_tile(idx_hbm, table, idx_v, ones_v, sid, n_items):
    per_tile = n_items // NS
    for c in range(per_tile // 256):
        off = sid * per_tile + c * 256
        pltpu.sync_copy(idx_hbm.at[pl.ds(off, 256)], idx_v)
        pltpu.sync_copy(ones_v, table.at[idx_v], add=True)


def _hist_body(col_hbm, node_hbm, he_hbm, ones_hbm, zeros_hbm,
               hc_out, hd_out, hb_out,
               table, idx_v, ones_v):
    cid = lax.axis_index("c")
    sid = lax.axis_index("s")
    pltpu.sync_copy(ones_hbm, ones_v)

    @pl.when(cid == 0)
    def _core0():
        _zero_table(zeros_hbm, table, sid, N_NODES_PAD // NS)
        plsc.subcore_barrier()
        _ones_scatter_tile(col_hbm, table, idx_v, ones_v, sid, N_EDGES)
        plsc.subcore_barrier()
        _writeback(table, hc_out, sid, N_NODES_PAD // NS)
        plsc.subcore_barrier()
        _zero_table(zeros_hbm, table, sid, N_EDGES // NS)
        plsc.subcore_barrier()
        _ones_scatter_tile(node_hbm, table, idx_v, ones_v, sid, HE_NNZ)
        plsc.subcore_barrier()
        _writeback(table, hd_out, sid, N_EDGES // NS)

    @pl.when(cid == 1)
    def _core1():
        _zero_table(zeros_hbm, table, sid, N_EDGES // NS)
        plsc.subcore_barrier()
        _ones_scatter_tile(he_hbm, table, idx_v, ones_v, sid, HE_NNZ)
        plsc.subcore_barrier()
        _writeback(table, hb_out, sid, N_EDGES // NS)


def _sc_hist(col_idx, node_idx, he_idx, ones128, zeros128):
    f32 = jnp.float32
    fn = pl.kernel(
        _hist_body,
        mesh=_sc_mesh(),
        out_type=[
            jax.ShapeDtypeStruct((N_NODES_PAD, FEAT), f32),
            jax.ShapeDtypeStruct((N_EDGES, FEAT), f32),
            jax.ShapeDtypeStruct((N_EDGES, FEAT), f32),
        ],
        scratch_types=[
            pltpu.VMEM_SHARED((N_NODES_PAD, FEAT), f32),
            pltpu.VMEM((256,), jnp.int32),
            pltpu.VMEM((256, FEAT), f32),
        ],
    )
    return fn(col_idx, node_idx, he_idx, ones128, zeros128)


def _pair_body(xs_hbm, g_hbm, row_hbm, col_hbm, gi2_hbm, si2_hbm, zeros_hbm,
               accg_out, acch_out,
               table, idx_v, rows_v, sem):
    cid = lax.axis_index("c")
    sid = lax.axis_index("s")

    @pl.when(cid == 0)
    def _core0():
        _zero_table(zeros_hbm, table, sid, N_NODES_PAD // NS)
        plsc.subcore_barrier()
        _scatter_tile(xs_hbm, row_hbm, col_hbm, table, idx_v, rows_v, sem,
                      sid, N_EDGES)
        plsc.subcore_barrier()
        _writeback(table, accg_out, sid, N_NODES_PAD // NS)

    @pl.when(cid == 1)
    def _core1():
        _zero_table(zeros_hbm, table, sid, N_EDGES // NS)
        plsc.subcore_barrier()
        _scatter_tile(g_hbm, gi2_hbm, si2_hbm, table, idx_v, rows_v, sem,
                      sid, HE_NNZ)
        plsc.subcore_barrier()
        _writeback(table, acch_out, sid, N_EDGES // NS)


def _sc_pair(xs, g, gather_idx1, scatter_idx1, gather_idx2, scatter_idx2,
             zeros128):
    f32 = jnp.float32
    fn = pl.kernel(
        _pair_body,
        mesh=_sc_mesh(),
        out_type=[
            jax.ShapeDtypeStruct((N_NODES_PAD, FEAT), f32),
            jax.ShapeDtypeStruct((N_EDGES, FEAT), f32),
        ],
        scratch_types=[
            pltpu.VMEM_SHARED((N_NODES_PAD, FEAT), f32),
            pltpu.VMEM((256,), jnp.int32),
            pltpu.VMEM((256, FEAT), f32),
            pltpu.SemaphoreType.DMA,
        ],
    )
    return fn(xs, g, gather_idx1, scatter_idx1, gather_idx2, scatter_idx2,
              zeros128)


def _single_body(src_hbm, gidx_hbm, sidx_hbm, p_hbm, q_hbm, row_hbm, col_hbm,
                 zeros_hbm,
                 acc_out, pg_out, qg_out,
                 table, idx_v, rows_v, sem):
    cid = lax.axis_index("c")
    sid = lax.axis_index("s")

    @pl.when(cid == 0)
    def _core0():
        _zero_table(zeros_hbm, table, sid, N_EDGES // NS)
        plsc.subcore_barrier()
        _scatter_tile(src_hbm, gidx_hbm, sidx_hbm, table, idx_v, rows_v, sem,
                      sid, HE_NNZ)
        plsc.subcore_barrier()
        _writeback(table, acc_out, sid, N_EDGES // NS)

    @pl.when(cid == 1)
    def _core1():
        for c in range(2):
            base = sid * 512 + c * 256
            pltpu.sync_copy(row_hbm.at[pl.ds(base, 256)], idx_v)
            pltpu.async_copy(p_hbm.at[idx_v], rows_v, sem).wait()
            pltpu.sync_copy(rows_v, pg_out.at[pl.ds(base, 256)])
            pltpu.sync_copy(col_hbm.at[pl.ds(base, 256)], idx_v)
            pltpu.async_copy(q_hbm.at[idx_v], rows_v, sem).wait()
            pltpu.sync_copy(rows_v, qg_out.at[pl.ds(base, 256)])


def _sc_hyp_and_gather(src, gidx, sidx, p, q, row_idx, col_idx, zeros128):
    f32 = jnp.float32
    fn = pl.kernel(
        _single_body,
        mesh=_sc_mesh(),
        out_type=[
            jax.ShapeDtypeStruct((N_EDGES, FEAT), f32),
            jax.ShapeDtypeStruct((N_EDGES, FEAT), f32),
            jax.ShapeDtypeStruct((N_EDGES, FEAT), f32),
        ],
        scratch_types=[
            pltpu.VMEM_SHARED((N_EDGES, FEAT), f32),
            pltpu.VMEM((256,), jnp.int32),
            pltpu.VMEM((256, FEAT), f32),
            pltpu.SemaphoreType.DMA,
        ],
    )
    return fn(src, gidx, sidx, p, q, row_idx, col_idx, zeros128)


def _last_body(src_hbm, gidx_hbm, sidx_hbm, zeros_hbm, acc_out,
               table, idx_v, rows_v, sem):
    cid = lax.axis_index("c")
    sid = lax.axis_index("s")
    wid = sid * NC + cid
    _zero_table(zeros_hbm, table, sid, N_EDGES // NS)
    plsc.subcore_barrier()
    for c in range(4):
        off = wid * 1024 + c * 256
        pltpu.sync_copy(gidx_hbm.at[pl.ds(off, 256)], idx_v)
        pltpu.async_copy(src_hbm.at[idx_v], rows_v, sem).wait()
        pltpu.sync_copy(sidx_hbm.at[pl.ds(off, 256)], idx_v)
        pltpu.sync_copy(rows_v, table.at[idx_v], add=True)
    plsc.subcore_barrier()
    r0 = sid * (N_EDGES // NS)
    pltpu.sync_copy(table.at[pl.ds(r0, N_EDGES // NS)],
                    acc_out.at[cid, pl.ds(r0, N_EDGES // NS)])


def _sc_hyp(src, gidx, sidx, zeros128):
    f32 = jnp.float32
    fn = pl.kernel(
        _last_body,
        mesh=_sc_mesh(),
        out_type=jax.ShapeDtypeStruct((NC, N_EDGES, FEAT), f32),
        scratch_types=[
            pltpu.VMEM_SHARED((N_EDGES, FEAT), f32),
            pltpu.VMEM((256,), jnp.int32),
            pltpu.VMEM((256, FEAT), f32),
            pltpu.SemaphoreType.DMA,
        ],
    )
    return fn(src, gidx, sidx, zeros128)


def _dinv_from_hist(hc_ref):
    h = hc_ref[:, 0] + 1.0
    return (1.0 / jnp.sqrt(h))[:N_NODES, None]


def _recip_from_hist(hr_ref):
    h = hr_ref[:, 0]
    return jnp.where(h > 0, 1.0 / h, 0.0)[:, None]


def _mm1_body(hc_ref, nodes_ref, w1_ref, edges_ref, wh_ref, xs_ref, g_ref):
    xw = jnp.dot(nodes_ref[...], w1_ref[...],
                 preferred_element_type=jnp.float32)
    xs_ref[...] = _dinv_from_hist(hc_ref) * xw
    g_ref[...] = jnp.dot(edges_ref[...], wh_ref[...],
                         preferred_element_type=jnp.float32)


def _tc_mm1(hc, nodes, w1, edges, wh):
    return pl.pallas_call(
        _mm1_body,
        out_shape=[jax.ShapeDtypeStruct((N_NODES, FEAT), jnp.float32),
                   jax.ShapeDtypeStruct((N_EDGES, FEAT), jnp.float32)],
    )(hc, nodes, w1, edges, wh)


def _l1fin_body(hc_ref, accg_ref, xs1_ref, b1_ref, w2_ref,
                hb_ref, acch_ref, xs2_ref, he1_ref):
    dinv = _dinv_from_hist(hc_ref)
    h = _leaky(dinv * (accg_ref[:N_NODES, :] + xs1_ref[...]) + b1_ref[...])
    xw2 = jnp.dot(h, w2_ref[...], preferred_element_type=jnp.float32)
    xs2_ref[...] = dinv * xw2
    he1_ref[...] = _recip_from_hist(hb_ref) * acch_ref[...]


def _tc_l1fin(hc, accg, xs1, b1, w2, hb, acch):
    return pl.pallas_call(
        _l1fin_body,
        out_shape=[jax.ShapeDtypeStruct((N_NODES, FEAT), jnp.float32),
                   jax.ShapeDtypeStruct((N_EDGES, FEAT), jnp.float32)],
    )(hc, accg, xs1, b1[None, :], w2, hb, acch)


def _l2fin_body(hc_ref, accg_ref, xs2_ref, b2_ref, nwa_ref, nwb_ref,
                hd_ref, acch_ref, hb1_ref, hw2_ref,
                p_ref, q_ref, gw2_ref):
    dinv = _dinv_from_hist(hc_ref)
    ne = _leaky(dinv * (accg_ref[:N_NODES, :] + xs2_ref[...]) + b2_ref[...])
    p_ref[...] = jnp.dot(ne, nwa_ref[...], preferred_element_type=jnp.float32)
    q_ref[...] = jnp.dot(ne, nwb_ref[...], preferred_element_type=jnp.float32)
    dinv_h = _recip_from_hist(hd_ref)
    g2 = _leaky(dinv_h * acch_ref[...] + hb1_ref[...])
    gw2_ref[...] = jnp.dot(g2, hw2_ref[...], preferred_element_type=jnp.float32)


def _tc_l2fin(hc, accg, xs2, b2, nwa, nwb, hd, acch, hb1, hw2):
    return pl.pallas_call(
        _l2fin_body,
        out_shape=[jax.ShapeDtypeStruct((N_NODES, FEAT), jnp.float32),
                   jax.ShapeDtypeStruct((N_NODES, FEAT), jnp.float32),
                   jax.ShapeDtypeStruct((N_EDGES, FEAT), jnp.float32)],
    )(hc, accg, xs2, b2[None, :], nwa, nwb, hd, acch, hb1[None, :], hw2)


def _he2_body(hb_ref, acch_ref, he2_ref):
    he2_ref[...] = _recip_from_hist(hb_ref) * acch_ref[...]


def _tc_he2(hb, acch):
    return pl.pallas_call(
        _he2_body,
        out_shape=jax.ShapeDtypeStruct((N_EDGES, FEAT), jnp.float32),
    )(hb, acch)


def _maps_body(hd_ref, acch_ref, hb2_ref, ew_ref, eb_ref,
               pg_ref, qg_ref, nb_ref, nm_ref, em_ref):
    dinv_h = _recip_from_hist(hd_ref)
    ee = _leaky(dinv_h * (acch_ref[0] + acch_ref[1]) + hb2_ref[...])
    emap = jnp.dot(ee, ew_ref[...], preferred_element_type=jnp.float32)
    emap = emap + eb_ref[...]
    nmap = (pg_ref[...] + qg_ref[...])[:, :MAP] + nb_ref[...]
    nm_ref[...] = nmap * lax.rsqrt(jnp.sum(nmap * nmap, axis=1,
                                           keepdims=True))
    em_ref[...] = emap * lax.rsqrt(jnp.sum(emap * emap, axis=1,
                                           keepdims=True))


def _tc_maps(hd, acch, hb2, ew, eb, pg, qg, nb):
    return pl.pallas_call(
        _maps_body,
        out_shape=[jax.ShapeDtypeStruct((N_EDGES, MAP), jnp.float32),
                   jax.ShapeDtypeStruct((N_EDGES, MAP), jnp.float32)],
    )(hd, acch, hb2[None, :], ew, eb[None, :], pg, qg, nb[None, :])


def _contrast_body(nm_ref, em_ref, out_ref, rs_ref, cs_ref, d_ref, *, bj, e):
    j = pl.program_id(0)
    nj = pl.num_programs(0)
    nm = nm_ref[...]
    em = em_ref[...]
    s = lax.dot_general(nm, em, (((1,), (1,)), ((), ())),
                        preferred_element_type=jnp.float32)
    z = jnp.exp(-jnp.abs(s))

    @pl.when(j == 0)
    def _init():
        rs_ref[...] = jnp.zeros_like(rs_ref)

    rs_ref[0, :] += jnp.sum(z, axis=1)
    cs_ref[0, pl.ds(j * bj, bj)] = jnp.sum(z, axis=0)

    nm_blk = nm_ref[pl.ds(j * bj, bj), :]
    d_ref[0, pl.ds(j * bj, bj)] = jnp.sum(nm_blk * em, axis=1)

    @pl.when(j == nj - 1)
    def _fin():
        out_ref[0, :] = (jnp.abs(d_ref[0, :]) - LOG2
                         + jnp.log(rs_ref[0, :] + cs_ref[0, :]))


def _contrast(nm, em, *, bj=512, interpret=False):
    e = nm.shape[0]
    nj = e // bj
    body = functools.partial(_contrast_body, bj=bj, e=e)
    out = pl.pallas_call(
        body,
        grid=(nj,),
        in_specs=[
            pl.BlockSpec((e, nm.shape[1]), lambda j: (0, 0)),
            pl.BlockSpec((bj, em.shape[1]), lambda j: (j, 0)),
        ],
        out_specs=pl.BlockSpec((1, e), lambda j: (0, 0)),
        out_shape=jax.ShapeDtypeStruct((1, e), jnp.float32),
        scratch_shapes=[
            pltpu.VMEM((1, e), jnp.float32),
            pltpu.VMEM((1, e), jnp.float32),
            pltpu.VMEM((1, e), jnp.float32),
        ],
        interpret=interpret,
    )(nm, em)
    return out[0]


def kernel(nodes_feature, edges_feature, edge_index, hyperedge_index,
           gcn_w1, gcn_b1, gcn_w2, gcn_b2,
           hgc_w1, hgc_b1, hgc_w2, hgc_b2,
           node_w, node_b, edge_w, edge_b):
    f32 = jnp.float32
    row_idx = edge_index[0]
    col_idx = edge_index[1]
    node_idx = hyperedge_index[0]
    he_idx = hyperedge_index[1]

    ones128 = jnp.ones((256, FEAT), f32)
    zeros128 = jnp.zeros((N_NODES_PAD, FEAT), f32)

    hc, hd, hb = _sc_hist(col_idx, node_idx, he_idx, ones128, zeros128)

    xs1, g1 = _tc_mm1(hc, nodes_feature, gcn_w1, edges_feature, hgc_w1)

    accg1, acch1a = _sc_pair(xs1, g1, row_idx, col_idx, node_idx, he_idx,
                             zeros128)

    xs2, he1 = _tc_l1fin(hc, accg1, xs1, gcn_b1, gcn_w2, hb, acch1a)

    accg2, acch1b = _sc_pair(xs2, he1, row_idx, col_idx, he_idx, node_idx,
                             zeros128)

    wpad = jnp.zeros((FEAT, FEAT - MAP), f32)
    nwa = jnp.concatenate([node_w[:FEAT], wpad], axis=1)
    nwb = jnp.concatenate([node_w[FEAT:], wpad], axis=1)
    p, q, gw2 = _tc_l2fin(hc, accg2, xs2, gcn_b2, nwa, nwb, hd, acch1b,
                          hgc_b1, hgc_w2)

    acch2a, pg, qg = _sc_hyp_and_gather(gw2, node_idx, he_idx, p, q,
                                        row_idx, col_idx, zeros128)

    he2 = _tc_he2(hb, acch2a)

    acch2b = _sc_hyp(he2, he_idx, node_idx, zeros128)

    nm, em = _tc_maps(hd, acch2b, hgc_b2, edge_w, edge_b, pg, qg, node_b)
    return _contrast(nm, em)


# speedup vs baseline: 4.8190x; 1.0629x over previous
import functools

import jax
import jax.numpy as jnp
from jax import lax
from jax.experimental import pallas as pl
from jax.experimental.pallas import tpu as pltpu
from jax.experimental.pallas import tpu_sc as plsc

N_NODES = 10000
N_NODES_PAD = 10112
N_EDGES = 8192
HE_NNZ = 32768
FEAT = 128
MAP = 64
LOG2 = 0.6931471805599453

NC = 2
NS = 16


def _leaky(x):
    return jnp.where(x >= 0, x, 0.01 * x)


def _sc_mesh():
    return plsc.VectorSubcoreMesh(core_axis_name="c", subcore_axis_name="s")


def _zero_table(zeros_hbm, table, sid, rows_per_tile):
    r0 = sid * rows_per_tile
    pltpu.sync_copy(zeros_hbm.at[pl.ds(r0, rows_per_tile)],
                    table.at[pl.ds(r0, rows_per_tile)])


def _writeback(table, out_hbm, sid, rows_per_tile):
    r0 = sid * rows_per_tile
    pltpu.sync_copy(table.at[pl.ds(r0, rows_per_tile)],
                    out_hbm.at[pl.ds(r0, rows_per_tile)])


def _scatter_tile(src_hbm, gidx_hbm, sidx_hbm, table, idx_v, rows_v, sem,
                  sid, n_items):
    per_tile = n_items // NS
    for c in range(per_tile // 256):
        off = sid * per_tile + c * 256
        pltpu.sync_copy(gidx_hbm.at[pl.ds(off, 256)], idx_v)
        pltpu.async_copy(src_hbm.at[idx_v], rows_v, sem).wait()
        pltpu.sync_copy(sidx_hbm.at[pl.ds(off, 256)], idx_v)
        pltpu.sync_copy(rows_v, table.at[idx_v], add=True)


def _ones_scatter_tile(idx_hbm, table, idx_v, ones_v, sid, n_items):
    per_tile = n_items // NS
    for c in range(per_tile // 256):
        off = sid * per_tile + c * 256
        pltpu.sync_copy(idx_hbm.at[pl.ds(off, 256)], idx_v)
        pltpu.sync_copy(ones_v, table.at[idx_v], add=True)


def _hist_body(col_hbm, node_hbm, he_hbm, ones_hbm, zeros_hbm,
               hc_out, hd_out, hb_out,
               table, idx_v, ones_v):
    cid = lax.axis_index("c")
    sid = lax.axis_index("s")
    pltpu.sync_copy(ones_hbm, ones_v)

    @pl.when(cid == 0)
    def _core0():
        _zero_table(zeros_hbm, table, sid, N_NODES_PAD // NS)
        plsc.subcore_barrier()
        _ones_scatter_tile(col_hbm, table, idx_v, ones_v, sid, N_EDGES)
        plsc.subcore_barrier()
        _writeback(table, hc_out, sid, N_NODES_PAD // NS)
        plsc.subcore_barrier()
        _zero_table(zeros_hbm, table, sid, N_EDGES // NS)
        plsc.subcore_barrier()
        _ones_scatter_tile(node_hbm, table, idx_v, ones_v, sid, HE_NNZ)
        plsc.subcore_barrier()
        _writeback(table, hd_out, sid, N_EDGES // NS)

    @pl.when(cid == 1)
    def _core1():
        _zero_table(zeros_hbm, table, sid, N_EDGES // NS)
        plsc.subcore_barrier()
        _ones_scatter_tile(he_hbm, table, idx_v, ones_v, sid, HE_NNZ)
        plsc.subcore_barrier()
        _writeback(table, hb_out, sid, N_EDGES // NS)


def _sc_hist(col_idx, node_idx, he_idx, ones128, zeros128):
    f32 = jnp.float32
    fn = pl.kernel(
        _hist_body,
        mesh=_sc_mesh(),
        out_type=[
            jax.ShapeDtypeStruct((N_NODES_PAD, FEAT), f32),
            jax.ShapeDtypeStruct((N_EDGES, FEAT), f32),
            jax.ShapeDtypeStruct((N_EDGES, FEAT), f32),
        ],
        scratch_types=[
            pltpu.VMEM_SHARED((N_NODES_PAD, FEAT), f32),
            pltpu.VMEM((256,), jnp.int32),
            pltpu.VMEM((256, FEAT), f32),
        ],
    )
    return fn(col_idx, node_idx, he_idx, ones128, zeros128)


def _pair_body(xs_hbm, g_hbm, row_hbm, col_hbm, gi2_hbm, si2_hbm, zeros_hbm,
               accg_out, acch_out,
               table, idx_v, rows_v, sem):
    cid = lax.axis_index("c")
    sid = lax.axis_index("s")

    @pl.when(cid == 0)
    def _core0():
        _zero_table(zeros_hbm, table, sid, N_NODES_PAD // NS)
        plsc.subcore_barrier()
        _scatter_tile(xs_hbm, row_hbm, col_hbm, table, idx_v, rows_v, sem,
                      sid, N_EDGES)
        plsc.subcore_barrier()
        _writeback(table, accg_out, sid, N_NODES_PAD // NS)

    @pl.when(cid == 1)
    def _core1():
        _zero_table(zeros_hbm, table, sid, N_EDGES // NS)
        plsc.subcore_barrier()
        _scatter_tile(g_hbm, gi2_hbm, si2_hbm, table, idx_v, rows_v, sem,
                      sid, HE_NNZ)
        plsc.subcore_barrier()
        _writeback(table, acch_out, sid, N_EDGES // NS)


def _sc_pair(xs, g, gather_idx1, scatter_idx1, gather_idx2, scatter_idx2,
             zeros128):
    f32 = jnp.float32
    fn = pl.kernel(
        _pair_body,
        mesh=_sc_mesh(),
        out_type=[
            jax.ShapeDtypeStruct((N_NODES_PAD, FEAT), f32),
            jax.ShapeDtypeStruct((N_EDGES, FEAT), f32),
        ],
        scratch_types=[
            pltpu.VMEM_SHARED((N_NODES_PAD, FEAT), f32),
            pltpu.VMEM((256,), jnp.int32),
            pltpu.VMEM((256, FEAT), f32),
            pltpu.SemaphoreType.DMA,
        ],
    )
    return fn(xs, g, gather_idx1, scatter_idx1, gather_idx2, scatter_idx2,
              zeros128)


def _single_body(src_hbm, gidx_hbm, sidx_hbm, p_hbm, q_hbm, row_hbm, col_hbm,
                 zeros_hbm,
                 acc_out, pg_out, qg_out,
                 table, idx_v, rows_v, sem):
    cid = lax.axis_index("c")
    sid = lax.axis_index("s")

    @pl.when(cid == 0)
    def _core0():
        _zero_table(zeros_hbm, table, sid, N_EDGES // NS)
        plsc.subcore_barrier()
        _scatter_tile(src_hbm, gidx_hbm, sidx_hbm, table, idx_v, rows_v, sem,
                      sid, HE_NNZ)
        plsc.subcore_barrier()
        _writeback(table, acc_out, sid, N_EDGES // NS)

    @pl.when(cid == 1)
    def _core1():
        for c in range(2):
            base = sid * 512 + c * 256
            pltpu.sync_copy(row_hbm.at[pl.ds(base, 256)], idx_v)
            pltpu.async_copy(p_hbm.at[idx_v], rows_v, sem).wait()
            pltpu.sync_copy(rows_v, pg_out.at[pl.ds(base, 256)])
            pltpu.sync_copy(col_hbm.at[pl.ds(base, 256)], idx_v)
            pltpu.async_copy(q_hbm.at[idx_v], rows_v, sem).wait()
            pltpu.sync_copy(rows_v, qg_out.at[pl.ds(base, 256)])


def _sc_hyp_and_gather(src, gidx, sidx, p, q, row_idx, col_idx, zeros128):
    f32 = jnp.float32
    fn = pl.kernel(
        _single_body,
        mesh=_sc_mesh(),
        out_type=[
            jax.ShapeDtypeStruct((N_EDGES, FEAT), f32),
            jax.ShapeDtypeStruct((N_EDGES, FEAT), f32),
            jax.ShapeDtypeStruct((N_EDGES, FEAT), f32),
        ],
        scratch_types=[
            pltpu.VMEM_SHARED((N_EDGES, FEAT), f32),
            pltpu.VMEM((256,), jnp.int32),
            pltpu.VMEM((256, FEAT), f32),
            pltpu.SemaphoreType.DMA,
        ],
    )
    return fn(src, gidx, sidx, p, q, row_idx, col_idx, zeros128)


def _last_body(src_hbm, gidx_hbm, sidx_hbm, zeros_hbm, acc_out,
               table, idx_v, rows_v, sem):
    cid = lax.axis_index("c")
    sid = lax.axis_index("s")
    wid = sid * NC + cid
    _zero_table(zeros_hbm, table, sid, N_EDGES // NS)
    plsc.subcore_barrier()
    for c in range(4):
        off = wid * 1024 + c * 256
        pltpu.sync_copy(gidx_hbm.at[pl.ds(off, 256)], idx_v)
        pltpu.async_copy(src_hbm.at[idx_v], rows_v, sem).wait()
        pltpu.sync_copy(sidx_hbm.at[pl.ds(off, 256)], idx_v)
        pltpu.sync_copy(rows_v, table.at[idx_v], add=True)
    plsc.subcore_barrier()
    r0 = sid * (N_EDGES // NS)
    pltpu.sync_copy(table.at[pl.ds(r0, N_EDGES // NS)],
                    acc_out.at[cid, pl.ds(r0, N_EDGES // NS)])


def _sc_hyp(src, gidx, sidx, zeros128):
    f32 = jnp.float32
    fn = pl.kernel(
        _last_body,
        mesh=_sc_mesh(),
        out_type=jax.ShapeDtypeStruct((NC, N_EDGES, FEAT), f32),
        scratch_types=[
            pltpu.VMEM_SHARED((N_EDGES, FEAT), f32),
            pltpu.VMEM((256,), jnp.int32),
            pltpu.VMEM((256, FEAT), f32),
            pltpu.SemaphoreType.DMA,
        ],
    )
    return fn(src, gidx, sidx, zeros128)


def _dinv_from_hist(hc_ref):
    h = hc_ref[:, 0] + 1.0
    return (1.0 / jnp.sqrt(h))[:N_NODES, None]


def _recip_from_hist(hr_ref):
    h = hr_ref[:, 0]
    return jnp.where(h > 0, 1.0 / h, 0.0)[:, None]


def _mm1_body(hc_ref, nodes_ref, w1_ref, edges_ref, wh_ref, xs_ref, g_ref):
    xw = jnp.dot(nodes_ref[...], w1_ref[...],
                 preferred_element_type=jnp.float32)
    xs_ref[...] = _dinv_from_hist(hc_ref) * xw
    g_ref[...] = jnp.dot(edges_ref[...], wh_ref[...],
                         preferred_element_type=jnp.float32)


def _tc_mm1(hc, nodes, w1, edges, wh):
    return pl.pallas_call(
        _mm1_body,
        out_shape=[jax.ShapeDtypeStruct((N_NODES, FEAT), jnp.float32),
                   jax.ShapeDtypeStruct((N_EDGES, FEAT), jnp.float32)],
    )(hc, nodes, w1, edges, wh)


def _l1fin_body(hc_ref, accg_ref, xs1_ref, b1_ref, w2_ref,
                hb_ref, acch_ref, xs2_ref, he1_ref):
    dinv = _dinv_from_hist(hc_ref)
    h = _leaky(dinv * (accg_ref[:N_NODES, :] + xs1_ref[...]) + b1_ref[...])
    xw2 = jnp.dot(h, w2_ref[...], preferred_element_type=jnp.float32)
    xs2_ref[...] = dinv * xw2
    he1_ref[...] = _recip_from_hist(hb_ref) * acch_ref[...]


def _tc_l1fin(hc, accg, xs1, b1, w2, hb, acch):
    return pl.pallas_call(
        _l1fin_body,
        out_shape=[jax.ShapeDtypeStruct((N_NODES, FEAT), jnp.float32),
                   jax.ShapeDtypeStruct((N_EDGES, FEAT), jnp.float32)],
    )(hc, accg, xs1, b1[None, :], w2, hb, acch)


def _l2fin_body(hc_ref, accg_ref, xs2_ref, b2_ref, nwa_ref, nwb_ref,
                hd_ref, acch_ref, hb1_ref, hw2_ref,
                p_ref, q_ref, gw2_ref):
    dinv = _dinv_from_hist(hc_ref)
    ne = _leaky(dinv * (accg_ref[:N_NODES, :] + xs2_ref[...]) + b2_ref[...])
    p_ref[...] = jnp.dot(ne, nwa_ref[...], preferred_element_type=jnp.float32)
    q_ref[...] = jnp.dot(ne, nwb_ref[...], preferred_element_type=jnp.float32)
    dinv_h = _recip_from_hist(hd_ref)
    g2 = _leaky(dinv_h * acch_ref[...] + hb1_ref[...])
    gw2_ref[...] = jnp.dot(g2, hw2_ref[...], preferred_element_type=jnp.float32)


def _tc_l2fin(hc, accg, xs2, b2, nwa, nwb, hd, acch, hb1, hw2):
    return pl.pallas_call(
        _l2fin_body,
        out_shape=[jax.ShapeDtypeStruct((N_NODES, FEAT), jnp.float32),
                   jax.ShapeDtypeStruct((N_NODES, FEAT), jnp.float32),
                   jax.ShapeDtypeStruct((N_EDGES, FEAT), jnp.float32)],
    )(hc, accg, xs2, b2[None, :], nwa, nwb, hd, acch, hb1[None, :], hw2)


def _he2_body(hb_ref, acch_ref, he2_ref):
    he2_ref[...] = _recip_from_hist(hb_ref) * acch_ref[...]


def _tc_he2(hb, acch):
    return pl.pallas_call(
        _he2_body,
        out_shape=jax.ShapeDtypeStruct((N_EDGES, FEAT), jnp.float32),
    )(hb, acch)


def _maps_body(hd_ref, acch_ref, hb2_ref, ew_ref, eb_ref,
               pg_ref, qg_ref, nb_ref, nm_ref, em_ref):
    dinv_h = _recip_from_hist(hd_ref)
    ee = _leaky(dinv_h * (acch_ref[0] + acch_ref[1]) + hb2_ref[...])
    emap = jnp.dot(ee, ew_ref[...], preferred_element_type=jnp.float32)
    emap = emap + eb_ref[...]
    nmap = (pg_ref[...] + qg_ref[...])[:, :MAP] + nb_ref[...]
    nm_ref[...] = nmap * lax.rsqrt(jnp.sum(nmap * nmap, axis=1,
                                           keepdims=True))
    em_ref[...] = emap * lax.rsqrt(jnp.sum(emap * emap, axis=1,
                                           keepdims=True))


def _tc_maps(hd, acch, hb2, ew, eb, pg, qg, nb):
    return pl.pallas_call(
        _maps_body,
        out_shape=[jax.ShapeDtypeStruct((N_EDGES, MAP), jnp.float32),
                   jax.ShapeDtypeStruct((N_EDGES, MAP), jnp.float32)],
    )(hd, acch, hb2[None, :], ew, eb[None, :], pg, qg, nb[None, :])


def _contrast_body(nm_ref, em_ref, out_ref, rs_ref, cs_ref, d_ref, *, bj, e):
    j = pl.program_id(0)
    nj = pl.num_programs(0)
    nm = nm_ref[...]
    em = em_ref[...]
    s = lax.dot_general(nm, em, (((1,), (1,)), ((), ())),
                        preferred_element_type=jnp.float32)
    z = jnp.exp(-jnp.abs(s))

    @pl.when(j == 0)
    def _init():
        rs_ref[...] = jnp.zeros_like(rs_ref)

    rs_ref[0, :] += jnp.sum(z, axis=1)
    cs_ref[0, pl.ds(j * bj, bj)] = jnp.sum(z, axis=0)

    nm_blk = nm_ref[pl.ds(j * bj, bj), :]
    d_ref[0, pl.ds(j * bj, bj)] = jnp.sum(nm_blk * em, axis=1)

    @pl.when(j == nj - 1)
    def _fin():
        out_ref[0, :] = (jnp.abs(d_ref[0, :]) - LOG2
                         + jnp.log(rs_ref[0, :] + cs_ref[0, :]))


def _contrast(nm, em, *, bj=1024, interpret=False):
    e = nm.shape[0]
    nj = e // bj
    body = functools.partial(_contrast_body, bj=bj, e=e)
    out = pl.pallas_call(
        body,
        grid=(nj,),
        in_specs=[
            pl.BlockSpec((e, nm.shape[1]), lambda j: (0, 0)),
            pl.BlockSpec((bj, em.shape[1]), lambda j: (j, 0)),
        ],
        out_specs=pl.BlockSpec((1, e), lambda j: (0, 0)),
        out_shape=jax.ShapeDtypeStruct((1, e), jnp.float32),
        scratch_shapes=[
            pltpu.VMEM((1, e), jnp.float32),
            pltpu.VMEM((1, e), jnp.float32),
            pltpu.VMEM((1, e), jnp.float32),
        ],
        interpret=interpret,
    )(nm, em)
    return out[0]


def kernel(nodes_feature, edges_feature, edge_index, hyperedge_index,
           gcn_w1, gcn_b1, gcn_w2, gcn_b2,
           hgc_w1, hgc_b1, hgc_w2, hgc_b2,
           node_w, node_b, edge_w, edge_b):
    f32 = jnp.float32
    row_idx = edge_index[0]
    col_idx = edge_index[1]
    node_idx = hyperedge_index[0]
    he_idx = hyperedge_index[1]

    ones128 = jnp.ones((256, FEAT), f32)
    zeros128 = jnp.zeros((N_NODES_PAD, FEAT), f32)

    hc, hd, hb = _sc_hist(col_idx, node_idx, he_idx, ones128, zeros128)

    xs1, g1 = _tc_mm1(hc, nodes_feature, gcn_w1, edges_feature, hgc_w1)

    accg1, acch1a = _sc_pair(xs1, g1, row_idx, col_idx, node_idx, he_idx,
                             zeros128)

    xs2, he1 = _tc_l1fin(hc, accg1, xs1, gcn_b1, gcn_w2, hb, acch1a)

    accg2, acch1b = _sc_pair(xs2, he1, row_idx, col_idx, he_idx, node_idx,
                             zeros128)

    wpad = jnp.zeros((FEAT, FEAT - MAP), f32)
    nwa = jnp.concatenate([node_w[:FEAT], wpad], axis=1)
    nwb = jnp.concatenate([node_w[FEAT:], wpad], axis=1)
    p, q, gw2 = _tc_l2fin(hc, accg2, xs2, gcn_b2, nwa, nwb, hd, acch1b,
                          hgc_b1, hgc_w2)

    acch2a, pg, qg = _sc_hyp_and_gather(gw2, node_idx, he_idx, p, q,
                                        row_idx, col_idx, zeros128)

    he2 = _tc_he2(hb, acch2a)

    acch2b = _sc_hyp(he2, he_idx, node_idx, zeros128)

    nm, em = _tc_maps(hd, acch2b, hgc_b2, edge_w, edge_b, pg, qg, node_b)
    return _contrast(nm, em)
